# baseline (speedup 1.0000x reference)
import jax
import jax.numpy as jnp
from jax.experimental import pallas as pl
from functools import partial


def _gnn_kernel(a_ref, x0_ref, w1_ref, b1_ref, w2_ref, b2_ref, w3_ref, b3_ref,
                out_ref):
    n = a_ref.shape[1]
    gpb = a_ref.shape[0]
    dn = (((0,), (0,)), ((), ()))
    y0 = jnp.dot(x0_ref[...], w1_ref[...], preferred_element_type=jnp.float32)

    aa = [a_ref[g] for g in range(gpb)]
    dinvs, u1s, h1s, h2s, ws = [], [], [], [], []
    for g in range(gpb):
        deg = jnp.sum(aa[g], axis=0).reshape(n, 1) + 1.0
        dinv = jnp.where(deg > 0.0, jax.lax.rsqrt(deg), 0.0)
        dinvs.append(dinv)
        u1s.append(dinv * y0)
        p1 = jax.lax.dot_general(aa[g], u1s[g], dn,
                                 preferred_element_type=jnp.float32)
        h1s.append(jax.nn.relu(dinv * (p1 + u1s[g]) + b1_ref[...]))
    u2s = [dinvs[g] * jnp.dot(h1s[g], w2_ref[...],
                              preferred_element_type=jnp.float32)
           for g in range(gpb)]
    for g in range(gpb):
        p2 = jax.lax.dot_general(aa[g], u2s[g], dn,
                                 preferred_element_type=jnp.float32)
        h2s.append(jax.nn.relu(dinvs[g] * (p2 + u2s[g]) + b2_ref[...]))
    for g in range(gpb):
        rs = jnp.dot(aa[g], dinvs[g], preferred_element_type=jnp.float32)
        ws.append(dinvs[g] * (rs + dinvs[g]))
    ts = [jnp.sum(ws[g] * h2s[g], axis=0, keepdims=True) for g in range(gpb)]
    t_all = jnp.concatenate(ts, axis=0)
    pooled = (jnp.dot(t_all, w3_ref[...], preferred_element_type=jnp.float32)
              / jnp.float32(n) + b3_ref[...])
    nrm = jnp.sqrt(jnp.sum(pooled * pooled, axis=1, keepdims=True))
    out_ref[...] = (pooled / jnp.maximum(nrm, 1e-12)).reshape(gpb, 1, -1)


@jax.jit
def kernel(adjacency_matrices, single_nodes, W1, b1, W2, b2, W3, b3):
    batch, n, _ = adjacency_matrices.shape
    out_c = W3.shape[1]
    gpb = 8

    def fixed(shape):
        return pl.BlockSpec(shape, lambda b: (0,) * len(shape))

    return pl.pallas_call(
        _gnn_kernel,
        grid=(batch // gpb,),
        in_specs=[
            pl.BlockSpec((gpb, n, n), lambda b: (b, 0, 0)),
            fixed(single_nodes.shape),
            fixed(W1.shape),
            fixed((1, b1.shape[0])),
            fixed(W2.shape),
            fixed((1, b2.shape[0])),
            fixed(W3.shape),
            fixed((1, b3.shape[0])),
        ],
        out_specs=pl.BlockSpec((gpb, 1, out_c), lambda b: (b, 0, 0)),
        out_shape=jax.ShapeDtypeStruct((batch, 1, out_c), jnp.float32),
    )(adjacency_matrices, single_nodes, W1, b1.reshape(1, -1),
      W2, b2.reshape(1, -1), W3, b3.reshape(1, -1)).reshape(batch, out_c)


# speedup vs baseline: 1.0527x; 1.0527x over previous
import jax
import jax.numpy as jnp
from jax.experimental import pallas as pl
from functools import partial


def _gnn_kernel(a_ref, x0_ref, w1_ref, b1_ref, w2_ref, b2_ref, w3_ref, b3_ref,
                out_ref):
    n = a_ref.shape[1]
    gpb = a_ref.shape[0]
    dn = (((0,), (0,)), ((), ()))
    y0 = jnp.dot(x0_ref[...], w1_ref[...], preferred_element_type=jnp.float32)

    aa = [a_ref[g] for g in range(gpb)]
    dinvs, u1s, h1s, h2s, ws = [], [], [], [], []
    for g in range(gpb):
        deg = jnp.sum(aa[g], axis=0).reshape(n, 1) + 1.0
        dinv = jnp.where(deg > 0.0, jax.lax.rsqrt(deg), 0.0)
        dinvs.append(dinv)
        u1s.append(dinv * y0)
        p1 = jax.lax.dot_general(aa[g], u1s[g], dn,
                                 preferred_element_type=jnp.float32)
        h1s.append(jax.nn.relu(dinv * (p1 + u1s[g]) + b1_ref[...]))
    u2s = [dinvs[g] * jnp.dot(h1s[g], w2_ref[...],
                              preferred_element_type=jnp.float32)
           for g in range(gpb)]
    for g in range(gpb):
        p2 = jax.lax.dot_general(aa[g], u2s[g], dn,
                                 preferred_element_type=jnp.float32)
        h2s.append(jax.nn.relu(dinvs[g] * (p2 + u2s[g]) + b2_ref[...]))
    for g in range(gpb):
        rs = jnp.dot(aa[g], dinvs[g], preferred_element_type=jnp.float32)
        ws.append(dinvs[g] * (rs + dinvs[g]))
    ts = [jnp.sum(ws[g] * h2s[g], axis=0, keepdims=True) for g in range(gpb)]
    t_all = jnp.concatenate(ts, axis=0)
    pooled = (jnp.dot(t_all, w3_ref[...], preferred_element_type=jnp.float32)
              / jnp.float32(n) + b3_ref[...])
    nrm = jnp.sqrt(jnp.sum(pooled * pooled, axis=1, keepdims=True))
    out_ref[...] = (pooled / jnp.maximum(nrm, 1e-12)).reshape(gpb, 1, -1)


@jax.jit
def kernel(adjacency_matrices, single_nodes, W1, b1, W2, b2, W3, b3):
    batch, n, _ = adjacency_matrices.shape
    out_c = W3.shape[1]
    gpb = 4

    def fixed(shape):
        return pl.BlockSpec(shape, lambda b: (0,) * len(shape))

    return pl.pallas_call(
        _gnn_kernel,
        grid=(batch // gpb,),
        in_specs=[
            pl.BlockSpec((gpb, n, n), lambda b: (b, 0, 0)),
            fixed(single_nodes.shape),
            fixed(W1.shape),
            fixed((1, b1.shape[0])),
            fixed(W2.shape),
            fixed((1, b2.shape[0])),
            fixed(W3.shape),
            fixed((1, b3.shape[0])),
        ],
        out_specs=pl.BlockSpec((gpb, 1, out_c), lambda b: (b, 0, 0)),
        out_shape=jax.ShapeDtypeStruct((batch, 1, out_c), jnp.float32),
    )(adjacency_matrices, single_nodes, W1, b1.reshape(1, -1),
      W2, b2.reshape(1, -1), W3, b3.reshape(1, -1)).reshape(batch, out_c)
